# Initial kernel scaffold; baseline (speedup 1.0000x reference)
import jax
import jax.numpy as jnp
from jax.experimental import pallas as pl


def kernel(x, edge_index, W, b):
    raise NotImplementedError("write your pallas kernel here")


# speedup vs baseline: 6.0158x; 6.0158x over previous
import functools

import jax
import jax.numpy as jnp
from jax import lax
from jax.experimental import pallas as pl
from jax.experimental.pallas import tpu as pltpu
from jax.experimental.pallas import tpu_sc as plsc

_N = 10000
_E = 320000
_D = 128
_OUT = 128

_NP = 10240
_NC = 2
_NS = 16
_NW = _NC * _NS
_EPW = _E // _NW
_C = 128
_NFULL = _EPW // _C
_REM = _EPW - _NFULL * _C
_RPT = _NP // _NS
_ZCH = 128
_CW = 16

_mesh = plsc.VectorSubcoreMesh(core_axis_name="c", subcore_axis_name="s")


def _zero_vmem(ref, nrows, ncols):
    z16 = jnp.zeros((16,), jnp.float32)

    @pl.loop(0, nrows)
    def _(i):
        for j in range(ncols // 16):
            ref[i, pl.ds(j * 16, 16)] = z16


def _seg_body(h_hbm, src_hbm, dst_hbm, out_hbm,
              acc, srcv, dstv, rows, srcv_r, dstv_r, rows_r, stage, sem):
    cid = lax.axis_index("c")
    sid = lax.axis_index("s")
    wid = sid * _NC + cid
    base_row = sid * _RPT

    _zero_vmem(stage, _ZCH, _D)
    for k in range(_RPT // _ZCH):
        pltpu.sync_copy(stage, acc.at[pl.ds(base_row + k * _ZCH, _ZCH)])
    plsc.subcore_barrier()

    ebase = wid * _EPW

    @pl.loop(0, _NFULL)
    def _(c):
        off = ebase + c * _C
        pltpu.sync_copy(src_hbm.at[pl.ds(off, _C)], srcv)
        pltpu.sync_copy(dst_hbm.at[pl.ds(off, _C)], dstv)
        pltpu.async_copy(h_hbm.at[srcv], rows, sem).wait()
        pltpu.sync_copy(rows, acc.at[dstv], add=True)

    if _REM:
        off = ebase + _NFULL * _C
        pltpu.sync_copy(src_hbm.at[pl.ds(off, _REM)], srcv_r)
        pltpu.sync_copy(dst_hbm.at[pl.ds(off, _REM)], dstv_r)
        pltpu.async_copy(h_hbm.at[srcv_r], rows_r, sem).wait()
        pltpu.sync_copy(rows_r, acc.at[dstv_r], add=True)

    plsc.subcore_barrier()

    for k in range(_RPT // _ZCH):
        r0 = base_row + k * _ZCH
        pltpu.sync_copy(acc.at[pl.ds(r0, _ZCH)], stage)
        pltpu.sync_copy(stage, out_hbm.at[cid, pl.ds(r0, _ZCH)])


_seg = pl.kernel(
    _seg_body,
    out_type=jax.ShapeDtypeStruct((_NC, _NP, _D), jnp.float32),
    mesh=_mesh,
    scratch_types=[
        pltpu.VMEM_SHARED((_NP, _D), jnp.float32),
        pltpu.VMEM((_C,), jnp.int32),
        pltpu.VMEM((_C,), jnp.int32),
        pltpu.VMEM((_C, _D), jnp.float32),
        pltpu.VMEM((_REM,), jnp.int32),
        pltpu.VMEM((_REM,), jnp.int32),
        pltpu.VMEM((_REM, _D), jnp.float32),
        pltpu.VMEM((_ZCH, _D), jnp.float32),
        pltpu.SemaphoreType.DMA,
    ],
)


def _cnt_body(dst_hbm, out_hbm, acc, dstv, dstv_r, ones, stage, sem):
    del sem
    cid = lax.axis_index("c")
    sid = lax.axis_index("s")
    wid = sid * _NC + cid
    base_row = sid * _RPT

    _zero_vmem(stage, _ZCH, _D)
    for k in range(_RPT // _ZCH):
        pltpu.sync_copy(stage, acc.at[pl.ds(base_row + k * _ZCH, _ZCH)])

    one16 = jnp.ones((16,), jnp.float32)

    @pl.loop(0, _C)
    def _(i):
        for j in range(_D // 16):
            ones[i, pl.ds(j * 16, 16)] = one16

    plsc.subcore_barrier()

    ebase = wid * _EPW

    @pl.loop(0, _NFULL)
    def _(c):
        off = ebase + c * _C
        pltpu.sync_copy(dst_hbm.at[pl.ds(off, _C)], dstv)
        pltpu.sync_copy(ones, acc.at[dstv], add=True)

    if _REM:
        off = ebase + _NFULL * _C
        pltpu.sync_copy(dst_hbm.at[pl.ds(off, _REM)], dstv_r)
        pltpu.sync_copy(ones.at[pl.ds(0, _REM)], acc.at[dstv_r], add=True)

    plsc.subcore_barrier()

    for k in range(_RPT // _ZCH):
        r0 = base_row + k * _ZCH
        pltpu.sync_copy(acc.at[pl.ds(r0, _ZCH)], stage)
        pltpu.sync_copy(stage, out_hbm.at[cid, pl.ds(r0, _ZCH)])


_cnt = pl.kernel(
    _cnt_body,
    out_type=jax.ShapeDtypeStruct((_NC, _NP, _D), jnp.float32),
    mesh=_mesh,
    scratch_types=[
        pltpu.VMEM_SHARED((_NP, _D), jnp.float32),
        pltpu.VMEM((_C,), jnp.int32),
        pltpu.VMEM((_REM,), jnp.int32),
        pltpu.VMEM((_C, _D), jnp.float32),
        pltpu.VMEM((_ZCH, _D), jnp.float32),
        pltpu.SemaphoreType.DMA,
    ],
)


_BR = 1024


def _fin1_kernel(x_ref, s_ref, c_ref, h_ref, inv_ref):
    cnt = c_ref[0, :, 0:1] + c_ref[1, :, 0:1]
    inv = 1.0 / jnp.maximum(cnt, 1.0)
    mean = (s_ref[0] + s_ref[1]) * inv
    h_ref[...] = jnp.maximum(0.5 * (x_ref[...] + mean), 0.0)
    inv_ref[...] = jnp.broadcast_to(inv, (_BR, _D))


def _fin2_kernel(h_ref, s_ref, inv_ref, o_ref):
    mean = (s_ref[0] + s_ref[1]) * inv_ref[...]
    h = h_ref[...]
    o_ref[...] = h + jnp.maximum(0.5 * (h + mean), 0.0)


def _out_kernel(h_ref, s_ref, inv_ref, w_ref, b_ref, o_ref):
    mean = (s_ref[0] + s_ref[1]) * inv_ref[...]
    h = h_ref[...]
    t = h + jnp.maximum(0.5 * (h + mean), 0.0)
    o_ref[...] = (
        jnp.dot(t, w_ref[...], preferred_element_type=jnp.float32,
                precision=lax.Precision.HIGHEST)
        + b_ref[...]
    )


_row_spec = pl.BlockSpec((_BR, _D), lambda i: (i, 0))
_part_spec = pl.BlockSpec((_NC, _BR, _D), lambda i: (0, i, 0))

_fin1 = pl.pallas_call(
    _fin1_kernel,
    grid=(_NP // _BR,),
    in_specs=[
        _row_spec,
        _part_spec,
        _part_spec,
    ],
    out_specs=[_row_spec, _row_spec],
    out_shape=[
        jax.ShapeDtypeStruct((_NP, _D), jnp.float32),
        jax.ShapeDtypeStruct((_NP, _D), jnp.float32),
    ],
)

_fin2 = pl.pallas_call(
    _fin2_kernel,
    grid=(_NP // _BR,),
    in_specs=[_row_spec, _part_spec, _row_spec],
    out_specs=_row_spec,
    out_shape=jax.ShapeDtypeStruct((_NP, _D), jnp.float32),
)

_outk = pl.pallas_call(
    _out_kernel,
    grid=(_NP // _BR,),
    in_specs=[
        _row_spec,
        _part_spec,
        _row_spec,
        pl.BlockSpec((_D, _OUT), lambda i: (0, 0)),
        pl.BlockSpec((1, _OUT), lambda i: (0, 0)),
    ],
    out_specs=pl.BlockSpec((_BR, _OUT), lambda i: (i, 0)),
    out_shape=jax.ShapeDtypeStruct((_NP, _OUT), jnp.float32),
)


def kernel(x, edge_index, W, b):
    src = edge_index[0]
    dst = edge_index[1]
    xp = jnp.pad(x, ((0, _NP - _N), (0, 0)))
    cnt_part = _cnt(dst)
    s1 = _seg(xp, src, dst)
    h1, inv = _fin1(xp, s1, cnt_part)
    s2 = _seg(h1, src, dst)
    h2 = _fin2(h1, s2, inv)
    s3 = _seg(h2, src, dst)
    return _outk(h2, s3, inv, W, b.reshape(1, _OUT))[:_N]


# speedup vs baseline: 7.4217x; 1.2337x over previous
import functools

import jax
import jax.numpy as jnp
from jax import lax
from jax.experimental import pallas as pl
from jax.experimental.pallas import tpu as pltpu
from jax.experimental.pallas import tpu_sc as plsc

_N = 10000
_E = 320000
_D = 128
_OUT = 128

_NP = 10240
_NC = 2
_NS = 16
_NW = _NC * _NS
_EPW = _E // _NW
_C = 128
_NFULL = _EPW // _C
_REM = _EPW - _NFULL * _C
_RPT = _NP // _NS
_ZCH = 64
_CW = 16

_mesh = plsc.VectorSubcoreMesh(core_axis_name="c", subcore_axis_name="s")


def _zero_vmem(ref, nrows, ncols):
    z16 = jnp.zeros((16,), jnp.float32)

    @pl.loop(0, nrows)
    def _(i):
        for j in range(ncols // 16):
            ref[i, pl.ds(j * 16, 16)] = z16


def _seg_body(h_hbm, src_hbm, dst_hbm, out_hbm, acc, stage, *rs):
    srcv = rs[0:2]
    dstv = rs[2:4]
    rows = rs[4:6]
    semg = rs[6:8]
    sems = rs[8:10]
    srcv_r, dstv_r, rows_r, sem_r = rs[10:14]

    cid = lax.axis_index("c")
    sid = lax.axis_index("s")
    wid = sid * _NC + cid
    base_row = sid * _RPT

    _zero_vmem(stage, _ZCH, _D)
    for k in range(_RPT // _ZCH):
        pltpu.sync_copy(stage, acc.at[pl.ds(base_row + k * _ZCH, _ZCH)])
    plsc.subcore_barrier()

    ebase = wid * _EPW

    def load_and_gather(off, s):
        pltpu.sync_copy(src_hbm.at[pl.ds(off, _C)], srcv[s])
        pltpu.sync_copy(dst_hbm.at[pl.ds(off, _C)], dstv[s])
        pltpu.async_copy(h_hbm.at[srcv[s]], rows[s], semg[s])

    def wait_gather(s):
        pltpu.make_async_copy(h_hbm.at[pl.ds(0, _C)], rows[s], semg[s]).wait()

    def issue_scatter(s):
        pltpu.async_copy(rows[s], acc.at[dstv[s]], sems[s], add=True)

    def wait_scatter(s):
        pltpu.make_async_copy(rows[s], acc.at[dstv[s]], sems[s]).wait()

    def step(c_off, b, wait_other, pf_off):
        wait_gather(b)
        issue_scatter(b)
        if pf_off is not None:
            if wait_other:
                wait_scatter(1 - b)
            load_and_gather(pf_off, 1 - b)

    load_and_gather(ebase, 0)
    step(ebase, 0, False, ebase + _C)
    step(ebase + _C, 1, True, ebase + 2 * _C)

    @pl.loop(1, _NFULL // 2 - 1)
    def _(i):
        off = ebase + 2 * i * _C
        step(off, 0, True, off + _C)
        step(off + _C, 1, True, off + 2 * _C)

    off = ebase + (_NFULL - 2) * _C
    step(off, 0, True, off + _C)
    step(off + _C, 1, False, None)

    for s in range(2):
        wait_scatter(s)

    if _REM:
        off = ebase + _NFULL * _C
        pltpu.sync_copy(src_hbm.at[pl.ds(off, _REM)], srcv_r)
        pltpu.sync_copy(dst_hbm.at[pl.ds(off, _REM)], dstv_r)
        pltpu.async_copy(h_hbm.at[srcv_r], rows_r, sem_r).wait()
        pltpu.sync_copy(rows_r, acc.at[dstv_r], add=True)

    plsc.subcore_barrier()

    for k in range(_RPT // _ZCH):
        r0 = base_row + k * _ZCH
        pltpu.sync_copy(acc.at[pl.ds(r0, _ZCH)], stage)
        pltpu.sync_copy(stage, out_hbm.at[cid, pl.ds(r0, _ZCH)])


_seg = pl.kernel(
    _seg_body,
    out_type=jax.ShapeDtypeStruct((_NC, _NP, _D), jnp.float32),
    mesh=_mesh,
    scratch_types=[
        pltpu.VMEM_SHARED((_NP, _D), jnp.float32),
        pltpu.VMEM((_ZCH, _D), jnp.float32),
        *[pltpu.VMEM((_C,), jnp.int32) for _ in range(2)],
        *[pltpu.VMEM((_C,), jnp.int32) for _ in range(2)],
        *[pltpu.VMEM((_C, _D), jnp.float32) for _ in range(2)],
        *[pltpu.SemaphoreType.DMA for _ in range(2)],
        *[pltpu.SemaphoreType.DMA for _ in range(2)],
        pltpu.VMEM((_REM,), jnp.int32),
        pltpu.VMEM((_REM,), jnp.int32),
        pltpu.VMEM((_REM, _D), jnp.float32),
        pltpu.SemaphoreType.DMA,
    ],
)


def _cnt_body(dst_hbm, out_hbm, acc, ones, stage, dstv0, dstv1, dstv_r,
              semc0, semc1):
    dstv = (dstv0, dstv1)
    semc = (semc0, semc1)
    cid = lax.axis_index("c")
    sid = lax.axis_index("s")
    wid = sid * _NC + cid
    base_row = sid * _RPT

    _zero_vmem(stage, _ZCH, _D)
    for k in range(_RPT // _ZCH):
        pltpu.sync_copy(stage, acc.at[pl.ds(base_row + k * _ZCH, _ZCH)])

    one16 = jnp.ones((16,), jnp.float32)

    @pl.loop(0, _C)
    def _(i):
        for j in range(_D // 16):
            ones[i, pl.ds(j * 16, 16)] = one16

    plsc.subcore_barrier()

    ebase = wid * _EPW

    def cload(off, s):
        pltpu.sync_copy(dst_hbm.at[pl.ds(off, _C)], dstv[s])

    def cissue(s):
        pltpu.async_copy(ones, acc.at[dstv[s]], semc[s], add=True)

    def cwait(s):
        pltpu.make_async_copy(ones, acc.at[dstv[s]], semc[s]).wait()

    for b in range(2):
        cload(ebase + b * _C, b)
        cissue(b)

    @pl.loop(1, _NFULL // 2)
    def _(i):
        for b in range(2):
            off = ebase + (2 * i + b) * _C
            cwait(b)
            cload(off, b)
            cissue(b)

    for b in range(2):
        cwait(b)

    if _REM:
        off = ebase + _NFULL * _C
        pltpu.sync_copy(dst_hbm.at[pl.ds(off, _REM)], dstv_r)
        pltpu.sync_copy(ones.at[pl.ds(0, _REM)], acc.at[dstv_r], add=True)

    plsc.subcore_barrier()

    for k in range(_RPT // _ZCH):
        r0 = base_row + k * _ZCH
        pltpu.sync_copy(acc.at[pl.ds(r0, _ZCH)], stage)
        pltpu.sync_copy(stage, out_hbm.at[cid, pl.ds(r0, _ZCH)])


_cnt = pl.kernel(
    _cnt_body,
    out_type=jax.ShapeDtypeStruct((_NC, _NP, _D), jnp.float32),
    mesh=_mesh,
    scratch_types=[
        pltpu.VMEM_SHARED((_NP, _D), jnp.float32),
        pltpu.VMEM((_C, _D), jnp.float32),
        pltpu.VMEM((_ZCH, _D), jnp.float32),
        pltpu.VMEM((_C,), jnp.int32),
        pltpu.VMEM((_C,), jnp.int32),
        pltpu.VMEM((_REM,), jnp.int32),
        pltpu.SemaphoreType.DMA,
        pltpu.SemaphoreType.DMA,
    ],
)


_BR = 1024


def _fin1_kernel(x_ref, s_ref, c_ref, h_ref, inv_ref):
    cnt = c_ref[0, :, 0:1] + c_ref[1, :, 0:1]
    inv = 1.0 / jnp.maximum(cnt, 1.0)
    mean = (s_ref[0] + s_ref[1]) * inv
    h_ref[...] = jnp.maximum(0.5 * (x_ref[...] + mean), 0.0)
    inv_ref[...] = jnp.broadcast_to(inv, (_BR, _D))


def _fin2_kernel(h_ref, s_ref, inv_ref, o_ref):
    mean = (s_ref[0] + s_ref[1]) * inv_ref[...]
    h = h_ref[...]
    o_ref[...] = h + jnp.maximum(0.5 * (h + mean), 0.0)


def _out_kernel(h_ref, s_ref, inv_ref, w_ref, b_ref, o_ref):
    mean = (s_ref[0] + s_ref[1]) * inv_ref[...]
    h = h_ref[...]
    t = h + jnp.maximum(0.5 * (h + mean), 0.0)
    o_ref[...] = (
        jnp.dot(t, w_ref[...], preferred_element_type=jnp.float32,
                precision=lax.Precision.HIGHEST)
        + b_ref[...]
    )


_row_spec = pl.BlockSpec((_BR, _D), lambda i: (i, 0))
_part_spec = pl.BlockSpec((_NC, _BR, _D), lambda i: (0, i, 0))

_fin1 = pl.pallas_call(
    _fin1_kernel,
    grid=(_NP // _BR,),
    in_specs=[
        _row_spec,
        _part_spec,
        _part_spec,
    ],
    out_specs=[_row_spec, _row_spec],
    out_shape=[
        jax.ShapeDtypeStruct((_NP, _D), jnp.float32),
        jax.ShapeDtypeStruct((_NP, _D), jnp.float32),
    ],
)

_fin2 = pl.pallas_call(
    _fin2_kernel,
    grid=(_NP // _BR,),
    in_specs=[_row_spec, _part_spec, _row_spec],
    out_specs=_row_spec,
    out_shape=jax.ShapeDtypeStruct((_NP, _D), jnp.float32),
)

_outk = pl.pallas_call(
    _out_kernel,
    grid=(_NP // _BR,),
    in_specs=[
        _row_spec,
        _part_spec,
        _row_spec,
        pl.BlockSpec((_D, _OUT), lambda i: (0, 0)),
        pl.BlockSpec((1, _OUT), lambda i: (0, 0)),
    ],
    out_specs=pl.BlockSpec((_BR, _OUT), lambda i: (i, 0)),
    out_shape=jax.ShapeDtypeStruct((_NP, _OUT), jnp.float32),
)


def kernel(x, edge_index, W, b):
    src = edge_index[0]
    dst = edge_index[1]
    xp = jnp.pad(x, ((0, _NP - _N), (0, 0)))
    cnt_part = _cnt(dst)
    s1 = _seg(xp, src, dst)
    h1, inv = _fin1(xp, s1, cnt_part)
    s2 = _seg(h1, src, dst)
    h2 = _fin2(h1, s2, inv)
    s3 = _seg(h2, src, dst)
    return _outk(h2, s3, inv, W, b.reshape(1, _OUT))[:_N]


# speedup vs baseline: 9.8612x; 1.3287x over previous
import functools

import jax
import jax.numpy as jnp
from jax import lax
from jax.experimental import pallas as pl
from jax.experimental.pallas import tpu as pltpu
from jax.experimental.pallas import tpu_sc as plsc

_N = 10000
_E = 320000
_D = 128
_OUT = 128

_NP = 10240
_NC = 2
_NS = 16
_NW = _NC * _NS
_C = 128
_ER = 2560
_GR = 8
_GPW = _ER // _NW // _GR
_NPAIR = _GPW // 2
_RPT = _NP // _NS
_ZCH = 64

_mesh = plsc.VectorSubcoreMesh(core_axis_name="c", subcore_axis_name="s")


def _zero_vmem(ref, nrows, ncols):
    z16 = jnp.zeros((16,), jnp.float32)

    @pl.loop(0, nrows)
    def _(i):
        for j in range(ncols // 16):
            ref[i, pl.ds(j * 16, 16)] = z16


def _seg_body(h_hbm, ei_hbm, out_hbm, acc, stage, idx0, idx1,
              rows0, rows1, semg0, semg1, sems0, sems1):
    idxg = (idx0, idx1)
    rows = (rows0, rows1)
    semg = (semg0, semg1)
    sems = (sems0, sems1)

    cid = lax.axis_index("c")
    sid = lax.axis_index("s")
    wid = sid * _NC + cid
    base_row = sid * _RPT

    _zero_vmem(stage, _ZCH, _D)
    for k in range(_RPT // _ZCH):
        pltpu.sync_copy(stage, acc.at[pl.ds(base_row + k * _ZCH, _ZCH)])
    plsc.subcore_barrier()

    gbase = wid * _GPW

    def load_idx(g, s):
        pltpu.sync_copy(ei_hbm.at[:, pl.ds((gbase + g) * _GR, _GR)], idxg[s])

    def gather(k, b, gs):
        pltpu.async_copy(h_hbm.at[idxg[gs].at[0, k]], rows[b], semg[b])

    def wait_gather(b):
        pltpu.make_async_copy(h_hbm.at[pl.ds(0, _C)], rows[b], semg[b]).wait()

    def scatter(k, b, gs):
        pltpu.async_copy(rows[b], acc.at[idxg[gs].at[1, k]], sems[b], add=True)

    def wait_scatter(b):
        pltpu.make_async_copy(rows[b], acc.at[idxg[0].at[1, 0]], sems[b]).wait()

    def pair(i, first, last):
        for h in range(2):
            for k in range(_GR):
                c = 16 * h + k
                b = k % 2
                wait_gather(b)
                scatter(k, b, h)
                if k == 3:
                    if h == 0:
                        load_idx(2 * i + 1, 1)
                    elif not last:
                        load_idx(2 * i + 2, 0)
                if last and h == 1 and k == 7:
                    continue
                kn = (k + 1) % _GR
                gsn = h if k < 7 else 1 - h
                if not (first and h == 0 and k == 0):
                    wait_scatter(1 - b)
                gather(kn, 1 - b, gsn)

    load_idx(0, 0)
    gather(0, 0, 0)
    pair(0, True, False)

    @pl.loop(1, _NPAIR - 1)
    def _(i):
        pair(i, False, False)

    pair(_NPAIR - 1, False, True)

    for b in range(2):
        wait_scatter(b)

    plsc.subcore_barrier()

    for k in range(_RPT // _ZCH):
        r0 = base_row + k * _ZCH
        pltpu.sync_copy(acc.at[pl.ds(r0, _ZCH)], stage)
        pltpu.sync_copy(stage, out_hbm.at[cid, pl.ds(r0, _ZCH)])


_seg = pl.kernel(
    _seg_body,
    out_type=jax.ShapeDtypeStruct((_NC, _NP, _D), jnp.float32),
    mesh=_mesh,
    scratch_types=[
        pltpu.VMEM_SHARED((_NP, _D), jnp.float32),
        pltpu.VMEM((_ZCH, _D), jnp.float32),
        *[pltpu.VMEM((2, _GR, _C), jnp.int32) for _ in range(2)],
        *[pltpu.VMEM((_C, _D), jnp.float32) for _ in range(2)],
        *[pltpu.SemaphoreType.DMA for _ in range(4)],
    ],
)


def _cnt_body(ei_hbm, out_hbm, acc, ones, stage, idx0, idx1, semc0, semc1):
    idxg = (idx0, idx1)
    semc = (semc0, semc1)
    cid = lax.axis_index("c")
    sid = lax.axis_index("s")
    wid = sid * _NC + cid
    base_row = sid * _RPT

    _zero_vmem(stage, _ZCH, _D)
    for k in range(_RPT // _ZCH):
        pltpu.sync_copy(stage, acc.at[pl.ds(base_row + k * _ZCH, _ZCH)])

    one16 = jnp.ones((16,), jnp.float32)

    @pl.loop(0, _C)
    def _(i):
        for j in range(_D // 16):
            ones[i, pl.ds(j * 16, 16)] = one16

    plsc.subcore_barrier()

    gbase = wid * _GPW

    def load_idx(g, s):
        pltpu.sync_copy(ei_hbm.at[1, pl.ds((gbase + g) * _GR, _GR)], idxg[s])

    def cissue(k, b, gs):
        pltpu.async_copy(ones, acc.at[idxg[gs].at[k]], semc[b], add=True)

    def cwait(b):
        pltpu.make_async_copy(ones, acc.at[idxg[0].at[0]], semc[b]).wait()

    def pair(i, first, last):
        for h in range(2):
            for k in range(_GR):
                b = k % 2
                if not (first and h == 0 and k < 2):
                    cwait(b)
                cissue(k, b, h)
                if k == 3:
                    if h == 0:
                        load_idx(2 * i + 1, 1)
                    elif not last:
                        load_idx(2 * i + 2, 0)

    load_idx(0, 0)
    pair(0, True, False)

    @pl.loop(1, _NPAIR - 1)
    def _(i):
        pair(i, False, False)

    pair(_NPAIR - 1, False, True)

    for b in range(2):
        cwait(b)

    plsc.subcore_barrier()

    for k in range(_RPT // _ZCH):
        r0 = base_row + k * _ZCH
        pltpu.sync_copy(acc.at[pl.ds(r0, _ZCH)], stage)
        pltpu.sync_copy(stage, out_hbm.at[cid, pl.ds(r0, _ZCH)])


_cnt = pl.kernel(
    _cnt_body,
    out_type=jax.ShapeDtypeStruct((_NC, _NP, _D), jnp.float32),
    mesh=_mesh,
    scratch_types=[
        pltpu.VMEM_SHARED((_NP, _D), jnp.float32),
        pltpu.VMEM((_C, _D), jnp.float32),
        pltpu.VMEM((_ZCH, _D), jnp.float32),
        pltpu.VMEM((_GR, _C), jnp.int32),
        pltpu.VMEM((_GR, _C), jnp.int32),
        pltpu.SemaphoreType.DMA,
        pltpu.SemaphoreType.DMA,
    ],
)


_BR = 1024


def _fin1_kernel(x_ref, s_ref, c_ref, h_ref, inv_ref):
    cnt = c_ref[0, :, 0:1] + c_ref[1, :, 0:1]
    inv = 1.0 / jnp.maximum(cnt, 1.0)
    mean = (s_ref[0] + s_ref[1]) * inv
    h_ref[...] = jnp.maximum(0.5 * (x_ref[...] + mean), 0.0)
    inv_ref[...] = jnp.broadcast_to(inv, (_BR, _D))


def _fin2_kernel(h_ref, s_ref, inv_ref, o_ref):
    mean = (s_ref[0] + s_ref[1]) * inv_ref[...]
    h = h_ref[...]
    o_ref[...] = h + jnp.maximum(0.5 * (h + mean), 0.0)


def _out_kernel(h_ref, s_ref, inv_ref, w_ref, b_ref, o_ref):
    mean = (s_ref[0] + s_ref[1]) * inv_ref[...]
    h = h_ref[...]
    t = h + jnp.maximum(0.5 * (h + mean), 0.0)
    o_ref[...] = (
        jnp.dot(t, w_ref[...], preferred_element_type=jnp.float32,
                precision=lax.Precision.HIGHEST)
        + b_ref[...]
    )


_row_spec = pl.BlockSpec((_BR, _D), lambda i: (i, 0))
_part_spec = pl.BlockSpec((_NC, _BR, _D), lambda i: (0, i, 0))

_fin1 = pl.pallas_call(
    _fin1_kernel,
    grid=(_NP // _BR,),
    in_specs=[
        _row_spec,
        _part_spec,
        _part_spec,
    ],
    out_specs=[_row_spec, _row_spec],
    out_shape=[
        jax.ShapeDtypeStruct((_NP, _D), jnp.float32),
        jax.ShapeDtypeStruct((_NP, _D), jnp.float32),
    ],
)

_fin2 = pl.pallas_call(
    _fin2_kernel,
    grid=(_NP // _BR,),
    in_specs=[_row_spec, _part_spec, _row_spec],
    out_specs=_row_spec,
    out_shape=jax.ShapeDtypeStruct((_NP, _D), jnp.float32),
)

_outk = pl.pallas_call(
    _out_kernel,
    grid=(_NP // _BR,),
    in_specs=[
        _row_spec,
        _part_spec,
        _row_spec,
        pl.BlockSpec((_D, _OUT), lambda i: (0, 0)),
        pl.BlockSpec((1, _OUT), lambda i: (0, 0)),
    ],
    out_specs=pl.BlockSpec((_BR, _OUT), lambda i: (i, 0)),
    out_shape=jax.ShapeDtypeStruct((_NP, _OUT), jnp.float32),
)


def kernel(x, edge_index, W, b):
    n_pad_e = _ER * _C - _E
    pad_idx = _N + (jnp.arange(n_pad_e, dtype=jnp.int32) % (_NP - _N))
    ei = jnp.concatenate(
        [edge_index.astype(jnp.int32),
         jnp.broadcast_to(pad_idx, (2, n_pad_e))], axis=1
    ).reshape(2, _ER, _C)
    xp = jnp.pad(x, ((0, _NP - _N), (0, 0)))
    cnt_part = _cnt(ei)
    s1 = _seg(xp, ei)
    h1, inv = _fin1(xp, s1, cnt_part)
    s2 = _seg(h1, ei)
    h2 = _fin2(h1, s2, inv)
    s3 = _seg(h2, ei)
    return _outk(h2, s3, inv, W, b.reshape(1, _OUT))[:_N]


# speedup vs baseline: 9.8615x; 1.0000x over previous
import functools

import jax
import jax.numpy as jnp
from jax import lax
from jax.experimental import pallas as pl
from jax.experimental.pallas import tpu as pltpu
from jax.experimental.pallas import tpu_sc as plsc

_N = 10000
_E = 320000
_D = 128
_OUT = 128

_NP = 10240
_NC = 2
_NS = 16
_NW = _NC * _NS
_C = 128
_ER = 2560
_GR = 8
_GPW = _ER // _NW // _GR
_NPAIR = _GPW // 2
_RPT = _NP // _NS
_ZCH = 64

_mesh = plsc.VectorSubcoreMesh(core_axis_name="c", subcore_axis_name="s")


def _zero_vmem(ref, nrows, ncols):
    z16 = jnp.zeros((16,), jnp.float32)

    @pl.loop(0, nrows)
    def _(i):
        for j in range(ncols // 16):
            ref[i, pl.ds(j * 16, 16)] = z16


def _seg_body(h_hbm, ei_hbm, out_hbm, acc, stage, idx0, idx1,
              rows0, rows1, semg0, semg1, sems0, sems1):
    idxg = (idx0, idx1)
    rows = (rows0, rows1)
    semg = (semg0, semg1)
    sems = (sems0, sems1)

    cid = lax.axis_index("c")
    sid = lax.axis_index("s")
    wid = sid * _NC + cid
    base_row = sid * _RPT

    _zero_vmem(stage, _ZCH, _D)
    for k in range(_RPT // _ZCH):
        pltpu.sync_copy(stage, acc.at[pl.ds(base_row + k * _ZCH, _ZCH)])
    plsc.subcore_barrier()

    gbase = wid * _GPW

    def load_idx(g, s):
        pltpu.sync_copy(ei_hbm.at[:, pl.ds((gbase + g) * _GR, _GR)], idxg[s])

    def gather(k, b, gs):
        pltpu.async_copy(h_hbm.at[idxg[gs].at[0, k]], rows[b], semg[b])

    def wait_gather(b):
        pltpu.make_async_copy(h_hbm.at[pl.ds(0, _C)], rows[b], semg[b]).wait()

    def scatter(k, b, gs):
        pltpu.async_copy(rows[b], acc.at[idxg[gs].at[1, k]], sems[b], add=True)

    def wait_scatter(b):
        pltpu.make_async_copy(rows[b], acc.at[idxg[0].at[1, 0]], sems[b]).wait()

    def pair(i, first, last):
        for h in range(2):
            for k in range(_GR):
                c = 16 * h + k
                b = k % 2
                wait_gather(b)
                scatter(k, b, h)
                if k == 3:
                    if h == 0:
                        load_idx(2 * i + 1, 1)
                    elif not last:
                        load_idx(2 * i + 2, 0)
                if last and h == 1 and k == 7:
                    continue
                kn = (k + 1) % _GR
                gsn = h if k < 7 else 1 - h
                if not (first and h == 0 and k == 0):
                    wait_scatter(1 - b)
                gather(kn, 1 - b, gsn)

    load_idx(0, 0)
    gather(0, 0, 0)
    pair(0, True, False)

    @pl.loop(1, _NPAIR - 1)
    def _(i):
        pair(i, False, False)

    pair(_NPAIR - 1, False, True)

    for b in range(2):
        wait_scatter(b)

    plsc.subcore_barrier()

    for k in range(_RPT // _ZCH):
        r0 = base_row + k * _ZCH
        pltpu.sync_copy(acc.at[pl.ds(r0, _ZCH)], stage)
        pltpu.sync_copy(stage, out_hbm.at[cid, pl.ds(r0, _ZCH)])


_seg = pl.kernel(
    _seg_body,
    out_type=jax.ShapeDtypeStruct((_NC, _NP, _D), jnp.float32),
    mesh=_mesh,
    scratch_types=[
        pltpu.VMEM_SHARED((_NP, _D), jnp.float32),
        pltpu.VMEM((_ZCH, _D), jnp.float32),
        *[pltpu.VMEM((2, _GR, _C), jnp.int32) for _ in range(2)],
        *[pltpu.VMEM((_C, _D), jnp.float32) for _ in range(2)],
        *[pltpu.SemaphoreType.DMA for _ in range(4)],
    ],
)


_CNTW = 128


def _cnt_body(ei_hbm, out_hbm, acc, ones, stage, idx0, idx1, semc0, semc1):
    idxg = (idx0, idx1)
    semc = (semc0, semc1)
    cid = lax.axis_index("c")
    sid = lax.axis_index("s")
    wid = sid * _NC + cid
    base_row = sid * _RPT

    _zero_vmem(stage, _ZCH, _CNTW)
    for k in range(_RPT // _ZCH):
        pltpu.sync_copy(stage, acc.at[pl.ds(base_row + k * _ZCH, _ZCH)])

    one16 = jnp.ones((16,), jnp.float32)

    @pl.loop(0, _C)
    def _(i):
        for j in range(_CNTW // 16):
            ones[i, pl.ds(j * 16, 16)] = one16

    plsc.subcore_barrier()

    gbase = wid * _GPW

    def load_idx(g, s):
        pltpu.sync_copy(ei_hbm.at[1, pl.ds((gbase + g) * _GR, _GR)], idxg[s])

    def cissue(k, b, gs):
        pltpu.async_copy(ones, acc.at[idxg[gs].at[k]], semc[b], add=True)

    def cwait(b):
        pltpu.make_async_copy(ones, acc.at[idxg[0].at[0]], semc[b]).wait()

    def pair(i, first, last):
        for h in range(2):
            for k in range(_GR):
                b = k % 2
                if not (first and h == 0 and k < 2):
                    cwait(b)
                cissue(k, b, h)
                if k == 3:
                    if h == 0:
                        load_idx(2 * i + 1, 1)
                    elif not last:
                        load_idx(2 * i + 2, 0)

    load_idx(0, 0)
    pair(0, True, False)

    @pl.loop(1, _NPAIR - 1)
    def _(i):
        pair(i, False, False)

    pair(_NPAIR - 1, False, True)

    for b in range(2):
        cwait(b)

    plsc.subcore_barrier()

    for k in range(_RPT // _ZCH):
        r0 = base_row + k * _ZCH
        pltpu.sync_copy(acc.at[pl.ds(r0, _ZCH)], stage)
        pltpu.sync_copy(stage, out_hbm.at[cid, pl.ds(r0, _ZCH)])


_cnt = pl.kernel(
    _cnt_body,
    out_type=jax.ShapeDtypeStruct((_NC, _NP, _CNTW), jnp.float32),
    mesh=_mesh,
    scratch_types=[
        pltpu.VMEM_SHARED((_NP, _CNTW), jnp.float32),
        pltpu.VMEM((_C, _CNTW), jnp.float32),
        pltpu.VMEM((_ZCH, _CNTW), jnp.float32),
        pltpu.VMEM((_GR, _C), jnp.int32),
        pltpu.VMEM((_GR, _C), jnp.int32),
        pltpu.SemaphoreType.DMA,
        pltpu.SemaphoreType.DMA,
    ],
)


_BR = 1024


def _fin1_kernel(x_ref, s_ref, c_ref, h_ref, inv_ref):
    cnt = c_ref[0, :, 0:1] + c_ref[1, :, 0:1]
    inv = 1.0 / jnp.maximum(cnt, 1.0)
    mean = (s_ref[0] + s_ref[1]) * inv
    h_ref[...] = jnp.maximum(0.5 * (x_ref[...] + mean), 0.0)
    inv_ref[...] = jnp.broadcast_to(inv, (_BR, _D))


def _fin2_kernel(h_ref, s_ref, inv_ref, o_ref):
    mean = (s_ref[0] + s_ref[1]) * inv_ref[...]
    h = h_ref[...]
    o_ref[...] = h + jnp.maximum(0.5 * (h + mean), 0.0)


def _out_kernel(h_ref, s_ref, inv_ref, w_ref, b_ref, o_ref):
    mean = (s_ref[0] + s_ref[1]) * inv_ref[...]
    h = h_ref[...]
    t = h + jnp.maximum(0.5 * (h + mean), 0.0)
    o_ref[...] = (
        jnp.dot(t, w_ref[...], preferred_element_type=jnp.float32,
                precision=lax.Precision.HIGHEST)
        + b_ref[...]
    )


_row_spec = pl.BlockSpec((_BR, _D), lambda i: (i, 0))
_part_spec = pl.BlockSpec((_NC, _BR, _D), lambda i: (0, i, 0))

_fin1 = pl.pallas_call(
    _fin1_kernel,
    grid=(_NP // _BR,),
    in_specs=[
        _row_spec,
        _part_spec,
        pl.BlockSpec((_NC, _BR, _CNTW), lambda i: (0, i, 0)),
    ],
    out_specs=[_row_spec, _row_spec],
    out_shape=[
        jax.ShapeDtypeStruct((_NP, _D), jnp.float32),
        jax.ShapeDtypeStruct((_NP, _D), jnp.float32),
    ],
)

_fin2 = pl.pallas_call(
    _fin2_kernel,
    grid=(_NP // _BR,),
    in_specs=[_row_spec, _part_spec, _row_spec],
    out_specs=_row_spec,
    out_shape=jax.ShapeDtypeStruct((_NP, _D), jnp.float32),
)

_outk = pl.pallas_call(
    _out_kernel,
    grid=(_NP // _BR,),
    in_specs=[
        _row_spec,
        _part_spec,
        _row_spec,
        pl.BlockSpec((_D, _OUT), lambda i: (0, 0)),
        pl.BlockSpec((1, _OUT), lambda i: (0, 0)),
    ],
    out_specs=pl.BlockSpec((_BR, _OUT), lambda i: (i, 0)),
    out_shape=jax.ShapeDtypeStruct((_NP, _OUT), jnp.float32),
)


def kernel(x, edge_index, W, b):
    n_pad_e = _ER * _C - _E
    pad_idx = _N + (jnp.arange(n_pad_e, dtype=jnp.int32) % (_NP - _N))
    ei = jnp.concatenate(
        [edge_index.astype(jnp.int32),
         jnp.broadcast_to(pad_idx, (2, n_pad_e))], axis=1
    ).reshape(2, _ER, _C)
    xp = jnp.pad(x, ((0, _NP - _N), (0, 0)))
    cnt_part = _cnt(ei)
    s1 = _seg(xp, ei)
    h1, inv = _fin1(xp, s1, cnt_part)
    s2 = _seg(h1, ei)
    h2 = _fin2(h1, s2, inv)
    s3 = _seg(h2, ei)
    return _outk(h2, s3, inv, W, b.reshape(1, _OUT))[:_N]


# speedup vs baseline: 10.0559x; 1.0197x over previous
import functools

import jax
import jax.numpy as jnp
from jax import lax
from jax.experimental import pallas as pl
from jax.experimental.pallas import tpu as pltpu
from jax.experimental.pallas import tpu_sc as plsc

_N = 10000
_E = 320000
_D = 128
_OUT = 128

_NP = 10240
_NC = 2
_NS = 16
_NW = _NC * _NS
_C = 128
_ER = 2560
_GR = 8
_GPW = _ER // _NW // _GR
_NPAIR = _GPW // 2
_RPT = _NP // _NS
_ZCH = 64

_mesh = plsc.VectorSubcoreMesh(core_axis_name="c", subcore_axis_name="s")


def _zero_vmem(ref, nrows, ncols):
    z16 = jnp.zeros((16,), jnp.float32)

    @pl.loop(0, nrows)
    def _(i):
        for j in range(ncols // 16):
            ref[i, pl.ds(j * 16, 16)] = z16


def _seg_body(h_hbm, ei_hbm, out_hbm, acc, idx0, idx1,
              rows0, rows1, semg0, semg1, sems0, sems1):
    idxg = (idx0, idx1)
    rows = (rows0, rows1)
    semg = (semg0, semg1)
    sems = (sems0, sems1)

    cid = lax.axis_index("c")
    sid = lax.axis_index("s")
    wid = sid * _NC + cid
    base_row = sid * _RPT

    _zero_vmem(rows0, _C, _D)
    for k in range(_RPT // _C):
        pltpu.async_copy(rows0, acc.at[pl.ds(base_row + k * _C, _C)], semg0)
    for k in range(_RPT // _C):
        pltpu.make_async_copy(rows0, acc.at[pl.ds(base_row, _C)], semg0).wait()
    plsc.subcore_barrier()

    gbase = wid * _GPW

    def load_idx(g, s):
        pltpu.sync_copy(ei_hbm.at[:, pl.ds((gbase + g) * _GR, _GR)], idxg[s])

    def gather(k, b, gs):
        pltpu.async_copy(h_hbm.at[idxg[gs].at[0, k]], rows[b], semg[b])

    def wait_gather(b):
        pltpu.make_async_copy(h_hbm.at[pl.ds(0, _C)], rows[b], semg[b]).wait()

    def scatter(k, b, gs):
        pltpu.async_copy(rows[b], acc.at[idxg[gs].at[1, k]], sems[b], add=True)

    def wait_scatter(b):
        pltpu.make_async_copy(rows[b], acc.at[idxg[0].at[1, 0]], sems[b]).wait()

    def pair(i, first, last):
        for h in range(2):
            for k in range(_GR):
                c = 16 * h + k
                b = k % 2
                wait_gather(b)
                scatter(k, b, h)
                if k == 3:
                    if h == 0:
                        load_idx(2 * i + 1, 1)
                    elif not last:
                        load_idx(2 * i + 2, 0)
                if last and h == 1 and k == 7:
                    continue
                kn = (k + 1) % _GR
                gsn = h if k < 7 else 1 - h
                if not (first and h == 0 and k == 0):
                    wait_scatter(1 - b)
                gather(kn, 1 - b, gsn)

    load_idx(0, 0)
    gather(0, 0, 0)
    pair(0, True, False)

    @pl.loop(1, _NPAIR - 1)
    def _(i):
        pair(i, False, False)

    pair(_NPAIR - 1, False, True)

    for b in range(2):
        wait_scatter(b)

    plsc.subcore_barrier()

    for k in range(_RPT // _C):
        b = k % 2
        if k >= 2:
            pltpu.make_async_copy(
                rows[b], out_hbm.at[cid, pl.ds(base_row, _C)], sems[b]).wait()
        r0 = base_row + k * _C
        pltpu.async_copy(acc.at[pl.ds(r0, _C)], rows[b], semg[b])
        pltpu.make_async_copy(acc.at[pl.ds(r0, _C)], rows[b], semg[b]).wait()
        pltpu.async_copy(rows[b], out_hbm.at[cid, pl.ds(r0, _C)], sems[b])
    for b in range(2):
        pltpu.make_async_copy(
            rows[b], out_hbm.at[cid, pl.ds(base_row, _C)], sems[b]).wait()


_seg = pl.kernel(
    _seg_body,
    out_type=jax.ShapeDtypeStruct((_NC, _NP, _D), jnp.float32),
    mesh=_mesh,
    scratch_types=[
        pltpu.VMEM_SHARED((_NP, _D), jnp.float32),
        *[pltpu.VMEM((2, _GR, _C), jnp.int32) for _ in range(2)],
        *[pltpu.VMEM((_C, _D), jnp.float32) for _ in range(2)],
        *[pltpu.SemaphoreType.DMA for _ in range(4)],
    ],
)


_CNTW = 128


def _cnt_body(ei_hbm, out_hbm, acc, ones, stage, idx0, idx1, semc0, semc1):
    idxg = (idx0, idx1)
    semc = (semc0, semc1)
    cid = lax.axis_index("c")
    sid = lax.axis_index("s")
    wid = sid * _NC + cid
    base_row = sid * _RPT

    _zero_vmem(stage, _ZCH, _CNTW)
    for k in range(_RPT // _ZCH):
        pltpu.async_copy(stage, acc.at[pl.ds(base_row + k * _ZCH, _ZCH)], semc0)
    for k in range(_RPT // _ZCH):
        pltpu.make_async_copy(stage, acc.at[pl.ds(base_row, _ZCH)], semc0).wait()

    one16 = jnp.ones((16,), jnp.float32)

    @pl.loop(0, _C)
    def _(i):
        for j in range(_CNTW // 16):
            ones[i, pl.ds(j * 16, 16)] = one16

    plsc.subcore_barrier()

    gbase = wid * _GPW

    def load_idx(g, s):
        pltpu.sync_copy(ei_hbm.at[1, pl.ds((gbase + g) * _GR, _GR)], idxg[s])

    def cissue(k, b, gs):
        pltpu.async_copy(ones, acc.at[idxg[gs].at[k]], semc[b], add=True)

    def cwait(b):
        pltpu.make_async_copy(ones, acc.at[idxg[0].at[0]], semc[b]).wait()

    def pair(i, first, last):
        for h in range(2):
            for k in range(_GR):
                b = k % 2
                if not (first and h == 0 and k < 2):
                    cwait(b)
                cissue(k, b, h)
                if k == 3:
                    if h == 0:
                        load_idx(2 * i + 1, 1)
                    elif not last:
                        load_idx(2 * i + 2, 0)

    load_idx(0, 0)
    pair(0, True, False)

    @pl.loop(1, _NPAIR - 1)
    def _(i):
        pair(i, False, False)

    pair(_NPAIR - 1, False, True)

    for b in range(2):
        cwait(b)

    plsc.subcore_barrier()

    def slot(b):
        return ones.at[pl.ds(b * _ZCH, _ZCH)]

    for k in range(_RPT // _ZCH):
        b = k % 2
        if k >= 2:
            pltpu.make_async_copy(
                slot(b), out_hbm.at[cid, pl.ds(base_row, _ZCH)],
                semc[b]).wait()
        r0 = base_row + k * _ZCH
        pltpu.async_copy(acc.at[pl.ds(r0, _ZCH)], slot(b), semc[b])
        pltpu.make_async_copy(acc.at[pl.ds(r0, _ZCH)], slot(b), semc[b]).wait()
        pltpu.async_copy(slot(b), out_hbm.at[cid, pl.ds(r0, _ZCH)], semc[b])
    for b in range(2):
        pltpu.make_async_copy(
            slot(b), out_hbm.at[cid, pl.ds(base_row, _ZCH)], semc[b]).wait()


_cnt = pl.kernel(
    _cnt_body,
    out_type=jax.ShapeDtypeStruct((_NC, _NP, _CNTW), jnp.float32),
    mesh=_mesh,
    scratch_types=[
        pltpu.VMEM_SHARED((_NP, _CNTW), jnp.float32),
        pltpu.VMEM((_C, _CNTW), jnp.float32),
        pltpu.VMEM((_ZCH, _CNTW), jnp.float32),
        pltpu.VMEM((_GR, _C), jnp.int32),
        pltpu.VMEM((_GR, _C), jnp.int32),
        pltpu.SemaphoreType.DMA,
        pltpu.SemaphoreType.DMA,
    ],
)


_BR = 1024


def _fin1_kernel(x_ref, s_ref, c_ref, h_ref, inv_ref):
    cnt = c_ref[0, :, 0:1] + c_ref[1, :, 0:1]
    inv = 1.0 / jnp.maximum(cnt, 1.0)
    mean = (s_ref[0] + s_ref[1]) * inv
    h_ref[...] = jnp.maximum(0.5 * (x_ref[...] + mean), 0.0)
    inv_ref[...] = jnp.broadcast_to(inv, (_BR, _D))


def _fin2_kernel(h_ref, s_ref, inv_ref, o_ref):
    mean = (s_ref[0] + s_ref[1]) * inv_ref[...]
    h = h_ref[...]
    o_ref[...] = h + jnp.maximum(0.5 * (h + mean), 0.0)


def _out_kernel(h_ref, s_ref, inv_ref, w_ref, b_ref, o_ref):
    mean = (s_ref[0] + s_ref[1]) * inv_ref[...]
    h = h_ref[...]
    t = h + jnp.maximum(0.5 * (h + mean), 0.0)
    o_ref[...] = (
        jnp.dot(t, w_ref[...], preferred_element_type=jnp.float32,
                precision=lax.Precision.HIGHEST)
        + b_ref[...]
    )


_row_spec = pl.BlockSpec((_BR, _D), lambda i: (i, 0))
_part_spec = pl.BlockSpec((_NC, _BR, _D), lambda i: (0, i, 0))

_fin1 = pl.pallas_call(
    _fin1_kernel,
    grid=(_NP // _BR,),
    in_specs=[
        _row_spec,
        _part_spec,
        pl.BlockSpec((_NC, _BR, _CNTW), lambda i: (0, i, 0)),
    ],
    out_specs=[_row_spec, _row_spec],
    out_shape=[
        jax.ShapeDtypeStruct((_NP, _D), jnp.float32),
        jax.ShapeDtypeStruct((_NP, _D), jnp.float32),
    ],
)

_fin2 = pl.pallas_call(
    _fin2_kernel,
    grid=(_NP // _BR,),
    in_specs=[_row_spec, _part_spec, _row_spec],
    out_specs=_row_spec,
    out_shape=jax.ShapeDtypeStruct((_NP, _D), jnp.float32),
)

_outk = pl.pallas_call(
    _out_kernel,
    grid=(_NP // _BR,),
    in_specs=[
        _row_spec,
        _part_spec,
        _row_spec,
        pl.BlockSpec((_D, _OUT), lambda i: (0, 0)),
        pl.BlockSpec((1, _OUT), lambda i: (0, 0)),
    ],
    out_specs=pl.BlockSpec((_BR, _OUT), lambda i: (i, 0)),
    out_shape=jax.ShapeDtypeStruct((_NP, _OUT), jnp.float32),
)


def kernel(x, edge_index, W, b):
    n_pad_e = _ER * _C - _E
    pad_idx = _N + (jnp.arange(n_pad_e, dtype=jnp.int32) % (_NP - _N))
    ei = jnp.concatenate(
        [edge_index.astype(jnp.int32),
         jnp.broadcast_to(pad_idx, (2, n_pad_e))], axis=1
    ).reshape(2, _ER, _C)
    xp = jnp.pad(x, ((0, _NP - _N), (0, 0)))
    cnt_part = _cnt(ei)
    s1 = _seg(xp, ei)
    h1, inv = _fin1(xp, s1, cnt_part)
    s2 = _seg(h1, ei)
    h2 = _fin2(h1, s2, inv)
    s3 = _seg(h2, ei)
    return _outk(h2, s3, inv, W, b.reshape(1, _OUT))[:_N]
